# baseline (speedup 1.0000x reference)
import functools

import jax
import jax.numpy as jnp
from jax import lax
from jax.experimental import pallas as pl
from jax.experimental.pallas import tpu as pltpu
from jax.experimental.pallas import tpu_sc as plsc

_H = 16
_THRESH = 0.05
_AUX_COEF = 0.01
_BLKG = 512

_pallas_call = pl.pallas_call


def _nt(a, b):
    return lax.dot_general(a, b, (((1,), (1,)), ((), ())),
                           preferred_element_type=jnp.float32)


def _nn(a, b):
    return lax.dot_general(a, b, (((1,), (0,)), ((), ())),
                           preferred_element_type=jnp.float32)


def _ln(x, g, b, eps=1e-5):
    m = jnp.mean(x, axis=-1, keepdims=True)
    v = jnp.mean((x - m) ** 2, axis=-1, keepdims=True)
    return (x - m) * lax.rsqrt(v + eps) * g + b


def _fold_body(Wq, wq_i, Wk, wk_i, Wv, wv_i, out_w, adapt_w,
               bq, bq_i, bk, bk_i, bv, bv_i, out_b, adapt_b,
               Aq, Ak, Av, Ao, cq, ck, cv, co):
    Aq[...] = _nn(wq_i[...], Wq[...])
    Ak[...] = _nn(wk_i[...], Wk[...])
    Av[...] = _nn(wv_i[...], Wv[...])
    Ao[...] = _nn(adapt_w[...], out_w[...])
    cq[...] = _nt(bq[...], wq_i[...]) + bq_i[...]
    ck[...] = _nt(bk[...], wk_i[...]) + bk_i[...]
    cv[...] = _nt(bv[...], wv_i[...]) + bv_i[...]
    co[...] = _nt(out_b[...], adapt_w[...]) + adapt_b[...]


def _proj_body(q_ref, kv_ref, Aq, Ak, Av, cq, ck, cv, q2, k2, v2):
    x = q_ref[...]
    y = kv_ref[...]
    q2[...] = _nt(x, Aq[...]) + cq[...]
    k2[...] = _nt(y, Ak[...]) + ck[...]
    v2[...] = _nt(y, Av[...]) + cv[...]


def _attn_body(q_ref, k_ref, v_ref, o_ref, *, scale):
    q = q_ref[0]
    k = k_ref[0]
    v = v_ref[0]
    s = _nt(q, k) * scale
    m = jnp.max(s, axis=-1, keepdims=True)
    p = jnp.exp(s - m)
    l = jnp.sum(p, axis=-1, keepdims=True)
    o_ref[0] = _nn(p, v) / l


def _post_body(ctx_ref, qin_ref, Ao, co, g_ref, b_ref, x_out, xbf_out):
    t = _nt(ctx_ref[...], Ao[...]) + co[...] + qin_ref[...]
    xn = _ln(t, g_ref[...], b_ref[...])
    x_out[...] = xn
    xbf_out[...] = xn.astype(jnp.bfloat16)


def _router_body(x_ref, gw_ref, tv_ref, ti_ref, cnt_ref, aux_ref, acc_ref,
                 *, nblk, ne, t_total):
    tb = pl.program_id(0)
    logits = _nn(x_ref[...], gw_ref[...])
    m = jnp.max(logits, axis=-1, keepdims=True)
    p = jnp.exp(logits - m)
    probs = p / jnp.sum(p, axis=-1, keepdims=True)
    idx = lax.broadcasted_iota(jnp.int32, probs.shape, 1)

    def pick(pr):
        v = jnp.max(pr, axis=-1, keepdims=True)
        i = jnp.min(jnp.where(pr == v, idx, ne), axis=-1, keepdims=True)
        return v, i

    v1, i1 = pick(probs)
    pm = jnp.where(idx == i1, -1.0, probs)
    v2, i2 = pick(pm)
    pm = jnp.where(idx == i2, -1.0, pm)
    v3, i3 = pick(pm)
    g2 = jnp.where(v2 >= _THRESH, v2, 0.0)
    g3 = jnp.where(v3 >= _THRESH, v3, 0.0)
    tv_ref[...] = jnp.concatenate([v1, g2, g3], axis=-1)
    ti_ref[...] = jnp.concatenate([i1, i2, i3], axis=-1)

    oh1 = jnp.where(idx == i1, 1.0, 0.0)
    oh123 = oh1 + jnp.where(idx == i2, 1.0, 0.0) + jnp.where(idx == i3, 1.0, 0.0)
    me_part = jnp.sum(probs, axis=0, keepdims=True)
    ce_part = jnp.sum(oh1, axis=0, keepdims=True)
    cnt_part = jnp.sum(oh123, axis=0, keepdims=True)

    @pl.when(tb == 0)
    def _():
        acc_ref[...] = jnp.zeros_like(acc_ref)

    acc_ref[0:1, :] += me_part
    acc_ref[1:2, :] += ce_part
    acc_ref[2:3, :] += cnt_part

    @pl.when(tb == nblk - 1)
    def _():
        me = acc_ref[0:1, :] / t_total
        ce = acc_ref[1:2, :] / t_total
        aux_ref[...] = _AUX_COEF * ne * jnp.sum(me * ce, axis=-1, keepdims=True)
        cnt_ref[...] = acc_ref[2:3, :]


def _pos_body(eid_ref, cnt_ref, pos_ref, b2e_ref, run_ref,
              *, ab, ne, blkg, nblkg):
    i = pl.program_id(0)
    cnt = cnt_ref[...]
    pc = jnp.ceil(cnt / blkg) * blkg
    lt = jnp.where(lax.broadcasted_iota(jnp.int32, (ne, ne), 0)
                   < lax.broadcasted_iota(jnp.int32, (ne, ne), 1), 1.0, 0.0)
    off = _nn(pc, lt)

    eid = eid_ref[...]
    oh = jnp.where(eid == lax.broadcasted_iota(jnp.int32, (ab, ne), 1),
                   1.0, 0.0)
    lts = jnp.where(lax.broadcasted_iota(jnp.int32, (ab, ab), 0)
                    > lax.broadcasted_iota(jnp.int32, (ab, ab), 1), 1.0, 0.0)
    excl = _nn(lts, oh)

    @pl.when(i == 0)
    def _():
        run_ref[...] = jnp.zeros_like(run_ref)
        bi = lax.broadcasted_iota(jnp.int32, (nblkg, 1), 0).astype(jnp.float32) * blkg
        ge = jnp.where(bi >= off, 1.0, 0.0)
        b2e_ref[...] = (jnp.sum(ge, axis=-1, keepdims=True) - 1.0).astype(jnp.int32)

    rank = excl + run_ref[...]
    posf = jnp.sum(oh * (off + rank), axis=-1, keepdims=True)
    pos_ref[...] = posf.astype(jnp.int32)
    run_ref[...] += jnp.sum(oh, axis=0, keepdims=True)


def _sc_scatter(pos, tok, gate, a_pad):
    a_tot = pos.shape[0]
    nrows = a_tot // 128
    rpt = nrows // 16
    stripe = a_pad // 16
    pos2 = pos
    tok2 = tok
    gate2 = gate
    mesh = plsc.VectorSubcoreMesh(core_axis_name="c", subcore_axis_name="s")

    @functools.partial(
        pl.kernel, mesh=mesh,
        compiler_params=pltpu.CompilerParams(needs_layout_passes=False),
        out_type=[jax.ShapeDtypeStruct((a_pad,), jnp.int32),
                  jax.ShapeDtypeStruct((a_pad,), jnp.float32)],
        scratch_types=[pltpu.VMEM((rpt * 128,), jnp.int32),
                       pltpu.VMEM((rpt * 128,), jnp.int32),
                       pltpu.VMEM((rpt * 128,), jnp.float32),
                       pltpu.VMEM((stripe,), jnp.int32),
                       pltpu.VMEM((stripe,), jnp.float32),
                       pltpu.VMEM_SHARED((a_pad,), jnp.int32),
                       pltpu.VMEM_SHARED((a_pad,), jnp.float32)])
    def k(pos_hbm, tok_hbm, gate_hbm, rows_hbm, gates_hbm,
          pos_v, tok_v, gate_v, zi_v, zf_v, rows_sh, gates_sh):
        c = lax.axis_index("c")
        s = lax.axis_index("s")

        @pl.when(c == 0)
        def _():
            def zero(i, _):
                zi_v[pl.ds(i * 16, 16)] = jnp.zeros((16,), jnp.int32)
                zf_v[pl.ds(i * 16, 16)] = jnp.zeros((16,), jnp.float32)
                return _

            lax.fori_loop(0, stripe // 16, zero, None)
            pltpu.sync_copy(pos_hbm.at[pl.ds(s * rpt * 128, rpt * 128)], pos_v)
            pltpu.sync_copy(tok_hbm.at[pl.ds(s * rpt * 128, rpt * 128)], tok_v)
            pltpu.sync_copy(gate_hbm.at[pl.ds(s * rpt * 128, rpt * 128)], gate_v)
            pltpu.sync_copy(zi_v, rows_sh.at[pl.ds(s * stripe, stripe)])
            pltpu.sync_copy(zf_v, gates_sh.at[pl.ds(s * stripe, stripe)])
            plsc.subcore_barrier()

            def scat(j, _):
                idx = pos_v.at[pl.ds(j * 128, 128)]
                pltpu.sync_copy(tok_v.at[pl.ds(j * 128, 128)],
                                rows_sh.at[idx], add=True)
                pltpu.sync_copy(gate_v.at[pl.ds(j * 128, 128)],
                                gates_sh.at[idx], add=True)
                return _

            lax.fori_loop(0, rpt, scat, None)
            plsc.subcore_barrier()
            pltpu.sync_copy(rows_sh.at[pl.ds(s * stripe, stripe)],
                            rows_hbm.at[pl.ds(s * stripe, stripe)])
            pltpu.sync_copy(gates_sh.at[pl.ds(s * stripe, stripe)],
                            gates_hbm.at[pl.ds(s * stripe, stripe)])

    return k(pos2, tok2, gate2)


def _sc_gather(table, idx):
    if table.dtype == jnp.bfloat16:
        n, d0 = table.shape
        t32 = lax.bitcast_convert_type(
            table.reshape(n, d0 // 2, 2), jnp.int32)
        out = _sc_gather(t32, idx)
        return lax.bitcast_convert_type(out, jnp.bfloat16).reshape(
            idx.shape[0], d0)
    m = idx.shape[0]
    d = table.shape[1]
    dt = table.dtype
    nw = 32
    rows_w = m // nw
    ch = rows_w // 16
    mesh = plsc.VectorSubcoreMesh(core_axis_name="c", subcore_axis_name="s")

    @functools.partial(
        pl.kernel, mesh=mesh,
        compiler_params=pltpu.CompilerParams(needs_layout_passes=False),
        out_type=jax.ShapeDtypeStruct((m, d), dt),
        scratch_types=[pltpu.VMEM((rows_w,), jnp.int32),
                       pltpu.VMEM((ch, d), dt),
                       pltpu.VMEM((ch, d), dt),
                       pltpu.SemaphoreType.DMA,
                       pltpu.SemaphoreType.DMA,
                       pltpu.SemaphoreType.DMA,
                       pltpu.SemaphoreType.DMA])
    def k(table_hbm, idx_hbm, out_hbm, idx_v, buf0, buf1, sg0, sg1, sw0, sw1):
        wid = lax.axis_index("s") * 2 + lax.axis_index("c")
        base = wid * rows_w
        pltpu.sync_copy(idx_hbm.at[pl.ds(base, rows_w)], idx_v)

        def body(j, _):
            c0 = 2 * j
            c1 = 2 * j + 1
            g0 = pltpu.async_copy(
                table_hbm.at[idx_v.at[pl.ds(c0 * ch, ch)]], buf0, sg0)
            g1 = pltpu.async_copy(
                table_hbm.at[idx_v.at[pl.ds(c1 * ch, ch)]], buf1, sg1)
            g0.wait()
            w0 = pltpu.async_copy(
                buf0, out_hbm.at[pl.ds(base + c0 * ch, ch)], sw0)
            g1.wait()
            w1 = pltpu.async_copy(
                buf1, out_hbm.at[pl.ds(base + c1 * ch, ch)], sw1)
            w0.wait()
            w1.wait()
            return _

        lax.fori_loop(0, 8, body, None)

    return k(table, idx)


def _gffn_body(b2e_ref, xs_ref, g_ref, w1_ref, b1_ref, w2_ref, b2_ref,
               out_ref, acc_ref, *, nfb):
    fb = pl.program_id(1)
    h = _nn(xs_ref[...], w1_ref[0]) + b1_ref[0]
    h = 0.5 * h * (1.0 + lax.erf(h * (2.0 ** -0.5)))
    y = _nn(h.astype(jnp.bfloat16), w2_ref[0]) + b2_ref[0] * jnp.float32(fb == 0)
    acc = jnp.where(fb == 0, y, acc_ref[...] + y)

    @pl.when(fb != nfb - 1)
    def _():
        acc_ref[...] = acc

    @pl.when(fb == nfb - 1)
    def _():
        out_ref[...] = (acc * g_ref[...]).astype(jnp.bfloat16)


def _fin_body(x_ref, y3_ref, g_ref, b_ref, o_ref):
    y3 = y3_ref[...].astype(jnp.float32)
    t = x_ref[...] + y3[0] + y3[1] + y3[2]
    o_ref[...] = _ln(t, g_ref[...], b_ref[...])


def kernel(query, key_value, Wq, bq, Wk, bk, Wv, bv, in_w, in_b, out_w, out_b,
           adapt_w, adapt_b, ln1_g, ln1_b, ln2_g, ln2_b, gate_w, e_w1, e_b1,
           e_w2, e_b2):
    B, S, E = query.shape
    H = _H
    dh = E // H
    T = B * S
    NE, _, FF = e_w1.shape
    f32 = jnp.float32
    TOPK = 3
    A = T * TOPK
    BLKG = min(_BLKG, T)
    A_PAD = A + NE * BLKG
    NBLKG = A_PAD // BLKG

    wq_i, wk_i, wv_i = in_w[:E], in_w[E:2 * E], in_w[2 * E:]
    bq_i, bk_i, bv_i = in_b[:E], in_b[E:2 * E], in_b[2 * E:]
    row = lambda v: v.reshape(1, E)

    ee = jax.ShapeDtypeStruct((E, E), f32)
    re = jax.ShapeDtypeStruct((1, E), f32)
    Aq, Ak, Av, Ao, cq, ck, cv, co = _pallas_call(
        _fold_body,
        out_shape=[ee, ee, ee, ee, re, re, re, re],
    )(Wq, wq_i, Wk, wk_i, Wv, wv_i, out_w, adapt_w,
      row(bq), row(bq_i), row(bk), row(bk_i), row(bv), row(bv_i),
      row(out_b), row(adapt_b))

    TB = min(512, T)
    nt = T // TB
    qf = query.reshape(T, E)
    kvf = key_value.reshape(T, E)
    blk_te = pl.BlockSpec((TB, E), lambda i: (i, 0))
    blk_ee = pl.BlockSpec((E, E), lambda i: (0, 0))
    blk_1e = pl.BlockSpec((1, E), lambda i: (0, 0))
    te = jax.ShapeDtypeStruct((T, E), f32)
    q2, k2, v2 = _pallas_call(
        _proj_body,
        grid=(nt,),
        in_specs=[blk_te, blk_te, blk_ee, blk_ee, blk_ee, blk_1e, blk_1e, blk_1e],
        out_specs=[blk_te, blk_te, blk_te],
        out_shape=[te, te, te],
    )(qf, kvf, Aq, Ak, Av, cq, ck, cv)

    def heads(x):
        return x.reshape(B, S, H, dh).transpose(0, 2, 1, 3).reshape(B * H, S, dh)

    QB = min(512, S)
    ctx = _pallas_call(
        functools.partial(_attn_body, scale=1.0 / (dh ** 0.5)),
        grid=(B * H, S // QB),
        in_specs=[pl.BlockSpec((1, QB, dh), lambda bh, qb: (bh, qb, 0)),
                  pl.BlockSpec((1, S, dh), lambda bh, qb: (bh, 0, 0)),
                  pl.BlockSpec((1, S, dh), lambda bh, qb: (bh, 0, 0))],
        out_specs=pl.BlockSpec((1, QB, dh), lambda bh, qb: (bh, qb, 0)),
        out_shape=jax.ShapeDtypeStruct((B * H, S, dh), f32),
    )(heads(q2), heads(k2), heads(v2))
    ctxf = ctx.reshape(B, H, S, dh).transpose(0, 2, 1, 3).reshape(T, E)

    x, x_bf = _pallas_call(
        _post_body,
        grid=(nt,),
        in_specs=[blk_te, blk_te, blk_ee, blk_1e, blk_1e, blk_1e],
        out_specs=[blk_te, blk_te],
        out_shape=[te, jax.ShapeDtypeStruct((T, E), jnp.bfloat16)],
    )(ctxf, qf, Ao, co, row(ln1_g), row(ln1_b))

    blk_t3 = pl.BlockSpec((TB, TOPK), lambda i: (i, 0))
    blk_11 = pl.BlockSpec((1, 1), lambda i: (0, 0))
    blk_1n = pl.BlockSpec((1, NE), lambda i: (0, 0))
    tv, ti, cnt, aux = _pallas_call(
        functools.partial(_router_body, nblk=nt, ne=NE, t_total=T),
        grid=(nt,),
        in_specs=[blk_te, pl.BlockSpec((E, NE), lambda i: (0, 0))],
        out_specs=[blk_t3, blk_t3, blk_1n, blk_11],
        out_shape=[jax.ShapeDtypeStruct((T, TOPK), f32),
                   jax.ShapeDtypeStruct((T, TOPK), jnp.int32),
                   jax.ShapeDtypeStruct((1, NE), f32),
                   jax.ShapeDtypeStruct((1, 1), f32)],
        scratch_shapes=[pltpu.VMEM((3, NE), f32)],
    )(x, gate_w)

    nab = max(1, A // 1024)
    AB = A // nab
    eids = ti.reshape(A, 1)
    pos, b2e = _pallas_call(
        functools.partial(_pos_body, ab=AB, ne=NE, blkg=BLKG, nblkg=NBLKG),
        grid=(nab,),
        in_specs=[pl.BlockSpec((AB, 1), lambda i: (i, 0)), blk_1n],
        out_specs=[pl.BlockSpec((AB, 1), lambda i: (i, 0)),
                   pl.BlockSpec((NBLKG, 1), lambda i: (0, 0))],
        out_shape=[jax.ShapeDtypeStruct((A, 1), jnp.int32),
                   jax.ShapeDtypeStruct((NBLKG, 1), jnp.int32)],
        scratch_shapes=[pltpu.VMEM((1, NE), f32)],
    )(eids, cnt)

    posf = pos.reshape(A)
    tok = jnp.arange(A, dtype=jnp.int32) // TOPK
    rows_sorted, gates_sorted = _sc_scatter(posf, tok, tv.reshape(A), A_PAD)
    xs = _sc_gather(x_bf, rows_sorted)

    FFB = min(1024, FF)
    grid_spec = pltpu.PrefetchScalarGridSpec(
        num_scalar_prefetch=1,
        grid=(NBLKG, FF // FFB),
        in_specs=[pl.BlockSpec((BLKG, E), lambda b, fb, m: (b, 0)),
                  pl.BlockSpec((BLKG, 1), lambda b, fb, m: (b, 0)),
                  pl.BlockSpec((1, E, FFB), lambda b, fb, m: (m[b], 0, fb)),
                  pl.BlockSpec((1, 1, FFB), lambda b, fb, m: (m[b], 0, fb)),
                  pl.BlockSpec((1, FFB, E), lambda b, fb, m: (m[b], fb, 0)),
                  pl.BlockSpec((1, 1, E), lambda b, fb, m: (m[b], 0, 0))],
        out_specs=pl.BlockSpec((BLKG, E), lambda b, fb, m: (b, 0)),
        scratch_shapes=[pltpu.VMEM((BLKG, E), f32)],
    )
    ys = _pallas_call(
        functools.partial(_gffn_body, nfb=FF // FFB),
        grid_spec=grid_spec,
        out_shape=jax.ShapeDtypeStruct((A_PAD, E), jnp.bfloat16),
    )(b2e.reshape(NBLKG), xs, gates_sorted.reshape(A_PAD, 1),
      e_w1.astype(jnp.bfloat16), e_b1.reshape(NE, 1, FF),
      e_w2.astype(jnp.bfloat16), e_b2.reshape(NE, 1, E))

    perm = pos.reshape(T, TOPK).T.reshape(A)
    y3 = _sc_gather(ys, perm).reshape(TOPK, T, E)

    out = _pallas_call(
        _fin_body,
        grid=(nt,),
        in_specs=[blk_te,
                  pl.BlockSpec((TOPK, TB, E), lambda i: (0, i, 0)),
                  blk_1e, blk_1e],
        out_specs=blk_te,
        out_shape=te,
    )(x, y3, row(ln2_g), row(ln2_b))

    return out.reshape(B, S, E), aux.reshape(())


# speedup vs baseline: 1.6408x; 1.6408x over previous
import functools

import jax
import jax.numpy as jnp
from jax import lax
from jax.experimental import pallas as pl
from jax.experimental.pallas import tpu as pltpu
from jax.experimental.pallas import tpu_sc as plsc

_H = 16
_THRESH = 0.05
_AUX_COEF = 0.01
_BLKG = 512

_pallas_call = pl.pallas_call


def _nt(a, b):
    return lax.dot_general(a, b, (((1,), (1,)), ((), ())),
                           preferred_element_type=jnp.float32)


def _nn(a, b):
    return lax.dot_general(a, b, (((1,), (0,)), ((), ())),
                           preferred_element_type=jnp.float32)


def _ln(x, g, b, eps=1e-5):
    m = jnp.mean(x, axis=-1, keepdims=True)
    v = jnp.mean((x - m) ** 2, axis=-1, keepdims=True)
    return (x - m) * lax.rsqrt(v + eps) * g + b


def _pack_bf(x):
    k = x.shape[-1] // 2
    lo = lax.bitcast_convert_type(x[:, :k].astype(jnp.bfloat16),
                                  jnp.uint16).astype(jnp.uint32)
    hi = lax.bitcast_convert_type(x[:, k:].astype(jnp.bfloat16),
                                  jnp.uint16).astype(jnp.uint32)
    return lax.bitcast_convert_type(lo | (hi << 16), jnp.int32)


def _unpack_bf(p):
    u = lax.bitcast_convert_type(p, jnp.uint32)
    lo = lax.bitcast_convert_type((u & 0xFFFF).astype(jnp.uint16), jnp.bfloat16)
    hi = lax.bitcast_convert_type((u >> 16).astype(jnp.uint16), jnp.bfloat16)
    return jnp.concatenate([lo, hi], axis=-1)


def _fold_body(Wq, wq_i, Wk, wk_i, Wv, wv_i, out_w, adapt_w,
               bq, bq_i, bk, bk_i, bv, bv_i, out_b, adapt_b,
               Aq, Ak, Av, Ao, cq, ck, cv, co):
    Aq[...] = _nn(wq_i[...], Wq[...])
    Ak[...] = _nn(wk_i[...], Wk[...])
    Av[...] = _nn(wv_i[...], Wv[...])
    Ao[...] = _nn(adapt_w[...], out_w[...])
    cq[...] = _nt(bq[...], wq_i[...]) + bq_i[...]
    ck[...] = _nt(bk[...], wk_i[...]) + bk_i[...]
    cv[...] = _nt(bv[...], wv_i[...]) + bv_i[...]
    co[...] = _nt(out_b[...], adapt_w[...]) + adapt_b[...]


def _proj_body(q_ref, kv_ref, Aq, Ak, Av, cq, ck, cv, q2, k2, v2):
    x = q_ref[...]
    y = kv_ref[...]
    q2[...] = _nt(x, Aq[...]) + cq[...]
    k2[...] = _nt(y, Ak[...]) + ck[...]
    v2[...] = _nt(y, Av[...]) + cv[...]


def _attn_body(q_ref, k_ref, v_ref, o_ref, *, scale):
    q = q_ref[0]
    k = k_ref[0]
    v = v_ref[0]
    s = _nt(q, k) * scale
    m = jnp.max(s, axis=-1, keepdims=True)
    p = jnp.exp(s - m)
    l = jnp.sum(p, axis=-1, keepdims=True)
    o_ref[0] = _nn(p, v) / l


def _post_body(ctx_ref, qin_ref, Ao, co, g_ref, b_ref, x_out, xpk_out):
    t = _nt(ctx_ref[...], Ao[...]) + co[...] + qin_ref[...]
    xn = _ln(t, g_ref[...], b_ref[...])
    x_out[...] = xn
    xpk_out[...] = _pack_bf(xn)


def _router_body(x_ref, gw_ref, tv_ref, ti_ref, cnt_ref, aux_ref, acc_ref,
                 *, nblk, ne, t_total):
    tb = pl.program_id(0)
    logits = _nn(x_ref[...], gw_ref[...])
    m = jnp.max(logits, axis=-1, keepdims=True)
    p = jnp.exp(logits - m)
    probs = p / jnp.sum(p, axis=-1, keepdims=True)
    idx = lax.broadcasted_iota(jnp.int32, probs.shape, 1)

    def pick(pr):
        v = jnp.max(pr, axis=-1, keepdims=True)
        i = jnp.min(jnp.where(pr == v, idx, ne), axis=-1, keepdims=True)
        return v, i

    v1, i1 = pick(probs)
    pm = jnp.where(idx == i1, -1.0, probs)
    v2, i2 = pick(pm)
    pm = jnp.where(idx == i2, -1.0, pm)
    v3, i3 = pick(pm)
    g2 = jnp.where(v2 >= _THRESH, v2, 0.0)
    g3 = jnp.where(v3 >= _THRESH, v3, 0.0)
    tv_ref[...] = jnp.concatenate([v1, g2, g3], axis=-1)
    ti_ref[...] = jnp.concatenate([i1, i2, i3], axis=-1)

    oh1 = jnp.where(idx == i1, 1.0, 0.0)
    oh123 = oh1 + jnp.where(idx == i2, 1.0, 0.0) + jnp.where(idx == i3, 1.0, 0.0)
    me_part = jnp.sum(probs, axis=0, keepdims=True)
    ce_part = jnp.sum(oh1, axis=0, keepdims=True)
    cnt_part = jnp.sum(oh123, axis=0, keepdims=True)

    @pl.when(tb == 0)
    def _():
        acc_ref[...] = jnp.zeros_like(acc_ref)

    acc_ref[0:1, :] += me_part
    acc_ref[1:2, :] += ce_part
    acc_ref[2:3, :] += cnt_part

    @pl.when(tb == nblk - 1)
    def _():
        me = acc_ref[0:1, :] / t_total
        ce = acc_ref[1:2, :] / t_total
        aux_ref[...] = _AUX_COEF * ne * jnp.sum(me * ce, axis=-1, keepdims=True)
        cnt_ref[...] = acc_ref[2:3, :]


def _pos_body(eid_ref, cnt_ref, pos_ref, b2e_ref, run_ref,
              *, ab, ne, blkg, nblkg):
    i = pl.program_id(0)
    cnt = cnt_ref[...]
    pc = jnp.ceil(cnt / blkg) * blkg
    lt = jnp.where(lax.broadcasted_iota(jnp.int32, (ne, ne), 0)
                   < lax.broadcasted_iota(jnp.int32, (ne, ne), 1), 1.0, 0.0)
    off = _nn(pc, lt)

    eid = eid_ref[...]
    oh = jnp.where(eid == lax.broadcasted_iota(jnp.int32, (ab, ne), 1),
                   1.0, 0.0)
    lts = jnp.where(lax.broadcasted_iota(jnp.int32, (ab, ab), 0)
                    > lax.broadcasted_iota(jnp.int32, (ab, ab), 1), 1.0, 0.0)
    excl = _nn(lts, oh)

    @pl.when(i == 0)
    def _():
        run_ref[...] = jnp.zeros_like(run_ref)
        bi = lax.broadcasted_iota(jnp.int32, (nblkg, 1), 0).astype(jnp.float32) * blkg
        ge = jnp.where(bi >= off, 1.0, 0.0)
        b2e_ref[...] = (jnp.sum(ge, axis=-1, keepdims=True) - 1.0).astype(jnp.int32)

    rank = excl + run_ref[...]
    posf = jnp.sum(oh * (off + rank), axis=-1, keepdims=True)
    pos_ref[...] = posf.astype(jnp.int32)
    run_ref[...] += jnp.sum(oh, axis=0, keepdims=True)


def _sc_scatter(pos, tok, gate, a_pad):
    a_tot = pos.shape[0]
    nrows = a_tot // 128
    rpt = nrows // 16
    stripe = a_pad // 16
    pos2 = pos
    tok2 = tok
    gate2 = gate
    mesh = plsc.VectorSubcoreMesh(core_axis_name="c", subcore_axis_name="s")

    @functools.partial(
        pl.kernel, mesh=mesh,
        compiler_params=pltpu.CompilerParams(needs_layout_passes=False),
        out_type=[jax.ShapeDtypeStruct((a_pad,), jnp.int32),
                  jax.ShapeDtypeStruct((a_pad,), jnp.float32)],
        scratch_types=[pltpu.VMEM((rpt * 128,), jnp.int32),
                       pltpu.VMEM((rpt * 128,), jnp.int32),
                       pltpu.VMEM((rpt * 128,), jnp.float32),
                       pltpu.VMEM((stripe,), jnp.int32),
                       pltpu.VMEM((stripe,), jnp.float32),
                       pltpu.VMEM_SHARED((a_pad,), jnp.int32),
                       pltpu.VMEM_SHARED((a_pad,), jnp.float32)])
    def k(pos_hbm, tok_hbm, gate_hbm, rows_hbm, gates_hbm,
          pos_v, tok_v, gate_v, zi_v, zf_v, rows_sh, gates_sh):
        c = lax.axis_index("c")
        s = lax.axis_index("s")

        @pl.when(c == 0)
        def _():
            def zero(i, _):
                zi_v[pl.ds(i * 16, 16)] = jnp.zeros((16,), jnp.int32)
                zf_v[pl.ds(i * 16, 16)] = jnp.zeros((16,), jnp.float32)
                return _

            lax.fori_loop(0, stripe // 16, zero, None)
            pltpu.sync_copy(pos_hbm.at[pl.ds(s * rpt * 128, rpt * 128)], pos_v)
            pltpu.sync_copy(tok_hbm.at[pl.ds(s * rpt * 128, rpt * 128)], tok_v)
            pltpu.sync_copy(gate_hbm.at[pl.ds(s * rpt * 128, rpt * 128)], gate_v)
            pltpu.sync_copy(zi_v, rows_sh.at[pl.ds(s * stripe, stripe)])
            pltpu.sync_copy(zf_v, gates_sh.at[pl.ds(s * stripe, stripe)])
            plsc.subcore_barrier()

            def scat(j, _):
                idx = pos_v.at[pl.ds(j * 128, 128)]
                pltpu.sync_copy(tok_v.at[pl.ds(j * 128, 128)],
                                rows_sh.at[idx], add=True)
                pltpu.sync_copy(gate_v.at[pl.ds(j * 128, 128)],
                                gates_sh.at[idx], add=True)
                return _

            lax.fori_loop(0, rpt, scat, None)
            plsc.subcore_barrier()
            pltpu.sync_copy(rows_sh.at[pl.ds(s * stripe, stripe)],
                            rows_hbm.at[pl.ds(s * stripe, stripe)])
            pltpu.sync_copy(gates_sh.at[pl.ds(s * stripe, stripe)],
                            gates_hbm.at[pl.ds(s * stripe, stripe)])

    return k(pos2, tok2, gate2)


def _sc_gather(table, idx):
    m = idx.shape[0]
    d = table.shape[1]
    dt = table.dtype
    nw = 32
    rows_w = m // nw
    ch = rows_w // 16
    mesh = plsc.VectorSubcoreMesh(core_axis_name="c", subcore_axis_name="s")

    @functools.partial(
        pl.kernel, mesh=mesh,
        compiler_params=pltpu.CompilerParams(needs_layout_passes=False),
        out_type=jax.ShapeDtypeStruct((m, d), dt),
        scratch_types=[pltpu.VMEM((rows_w,), jnp.int32),
                       pltpu.VMEM((ch, d), dt),
                       pltpu.VMEM((ch, d), dt),
                       pltpu.SemaphoreType.DMA,
                       pltpu.SemaphoreType.DMA,
                       pltpu.SemaphoreType.DMA,
                       pltpu.SemaphoreType.DMA])
    def k(table_hbm, idx_hbm, out_hbm, idx_v, buf0, buf1, sg0, sg1, sw0, sw1):
        wid = lax.axis_index("s") * 2 + lax.axis_index("c")
        base = wid * rows_w
        pltpu.sync_copy(idx_hbm.at[pl.ds(base, rows_w)], idx_v)

        def body(j, _):
            c0 = 2 * j
            c1 = 2 * j + 1
            g0 = pltpu.async_copy(
                table_hbm.at[idx_v.at[pl.ds(c0 * ch, ch)]], buf0, sg0)
            g1 = pltpu.async_copy(
                table_hbm.at[idx_v.at[pl.ds(c1 * ch, ch)]], buf1, sg1)
            g0.wait()
            w0 = pltpu.async_copy(
                buf0, out_hbm.at[pl.ds(base + c0 * ch, ch)], sw0)
            g1.wait()
            w1 = pltpu.async_copy(
                buf1, out_hbm.at[pl.ds(base + c1 * ch, ch)], sw1)
            w0.wait()
            w1.wait()
            return _

        lax.fori_loop(0, 8, body, None)

    return k(table, idx)


def _gffn_body(b2e_ref, xs_ref, g_ref, w1_ref, b1_ref, w2_ref, b2_ref,
               out_ref, acc_ref, *, nfb):
    fb = pl.program_id(1)
    xb = _unpack_bf(xs_ref[...])
    h = _nn(xb, w1_ref[0].astype(jnp.bfloat16)) + b1_ref[0]
    h = 0.5 * h * (1.0 + lax.erf(h * (2.0 ** -0.5)))
    y = (_nn(h.astype(jnp.bfloat16), w2_ref[0].astype(jnp.bfloat16))
         + b2_ref[0] * jnp.float32(fb == 0))
    acc = jnp.where(fb == 0, y, acc_ref[...] + y)

    @pl.when(fb != nfb - 1)
    def _():
        acc_ref[...] = acc

    @pl.when(fb == nfb - 1)
    def _():
        out_ref[...] = _pack_bf(acc * g_ref[...])


def _fin_body(x_ref, y3_ref, g_ref, b_ref, o_ref):
    y = (_unpack_bf(y3_ref[0]).astype(jnp.float32)
         + _unpack_bf(y3_ref[1]).astype(jnp.float32)
         + _unpack_bf(y3_ref[2]).astype(jnp.float32))
    t = x_ref[...] + y
    o_ref[...] = _ln(t, g_ref[...], b_ref[...])


def kernel(query, key_value, Wq, bq, Wk, bk, Wv, bv, in_w, in_b, out_w, out_b,
           adapt_w, adapt_b, ln1_g, ln1_b, ln2_g, ln2_b, gate_w, e_w1, e_b1,
           e_w2, e_b2):
    B, S, E = query.shape
    H = _H
    dh = E // H
    T = B * S
    NE, _, FF = e_w1.shape
    f32 = jnp.float32
    TOPK = 3
    A = T * TOPK
    BLKG = min(_BLKG, T)
    A_PAD = A + NE * BLKG
    NBLKG = A_PAD // BLKG

    wq_i, wk_i, wv_i = in_w[:E], in_w[E:2 * E], in_w[2 * E:]
    bq_i, bk_i, bv_i = in_b[:E], in_b[E:2 * E], in_b[2 * E:]
    row = lambda v: v.reshape(1, E)

    ee = jax.ShapeDtypeStruct((E, E), f32)
    re = jax.ShapeDtypeStruct((1, E), f32)
    Aq, Ak, Av, Ao, cq, ck, cv, co = _pallas_call(
        _fold_body,
        out_shape=[ee, ee, ee, ee, re, re, re, re],
    )(Wq, wq_i, Wk, wk_i, Wv, wv_i, out_w, adapt_w,
      row(bq), row(bq_i), row(bk), row(bk_i), row(bv), row(bv_i),
      row(out_b), row(adapt_b))

    TB = min(512, T)
    nt = T // TB
    qf = query.reshape(T, E)
    kvf = key_value.reshape(T, E)
    blk_te = pl.BlockSpec((TB, E), lambda i: (i, 0))
    blk_ee = pl.BlockSpec((E, E), lambda i: (0, 0))
    blk_1e = pl.BlockSpec((1, E), lambda i: (0, 0))
    te = jax.ShapeDtypeStruct((T, E), f32)
    q2, k2, v2 = _pallas_call(
        _proj_body,
        grid=(nt,),
        in_specs=[blk_te, blk_te, blk_ee, blk_ee, blk_ee, blk_1e, blk_1e, blk_1e],
        out_specs=[blk_te, blk_te, blk_te],
        out_shape=[te, te, te],
    )(qf, kvf, Aq, Ak, Av, cq, ck, cv)

    def heads(x):
        return x.reshape(B, S, H, dh).transpose(0, 2, 1, 3).reshape(B * H, S, dh)

    QB = min(512, S)
    ctx = _pallas_call(
        functools.partial(_attn_body, scale=1.0 / (dh ** 0.5)),
        grid=(B * H, S // QB),
        in_specs=[pl.BlockSpec((1, QB, dh), lambda bh, qb: (bh, qb, 0)),
                  pl.BlockSpec((1, S, dh), lambda bh, qb: (bh, 0, 0)),
                  pl.BlockSpec((1, S, dh), lambda bh, qb: (bh, 0, 0))],
        out_specs=pl.BlockSpec((1, QB, dh), lambda bh, qb: (bh, qb, 0)),
        out_shape=jax.ShapeDtypeStruct((B * H, S, dh), f32),
    )(heads(q2), heads(k2), heads(v2))
    ctxf = ctx.reshape(B, H, S, dh).transpose(0, 2, 1, 3).reshape(T, E)

    blk_th = pl.BlockSpec((TB, E // 2), lambda i: (i, 0))
    x, xpk = _pallas_call(
        _post_body,
        grid=(nt,),
        in_specs=[blk_te, blk_te, blk_ee, blk_1e, blk_1e, blk_1e],
        out_specs=[blk_te, blk_th],
        out_shape=[te, jax.ShapeDtypeStruct((T, E // 2), jnp.int32)],
    )(ctxf, qf, Ao, co, row(ln1_g), row(ln1_b))

    blk_t3 = pl.BlockSpec((TB, TOPK), lambda i: (i, 0))
    blk_11 = pl.BlockSpec((1, 1), lambda i: (0, 0))
    blk_1n = pl.BlockSpec((1, NE), lambda i: (0, 0))
    tv, ti, cnt, aux = _pallas_call(
        functools.partial(_router_body, nblk=nt, ne=NE, t_total=T),
        grid=(nt,),
        in_specs=[blk_te, pl.BlockSpec((E, NE), lambda i: (0, 0))],
        out_specs=[blk_t3, blk_t3, blk_1n, blk_11],
        out_shape=[jax.ShapeDtypeStruct((T, TOPK), f32),
                   jax.ShapeDtypeStruct((T, TOPK), jnp.int32),
                   jax.ShapeDtypeStruct((1, NE), f32),
                   jax.ShapeDtypeStruct((1, 1), f32)],
        scratch_shapes=[pltpu.VMEM((3, NE), f32)],
    )(x, gate_w)

    nab = max(1, A // 1024)
    AB = A // nab
    eids = ti.reshape(A, 1)
    pos, b2e = _pallas_call(
        functools.partial(_pos_body, ab=AB, ne=NE, blkg=BLKG, nblkg=NBLKG),
        grid=(nab,),
        in_specs=[pl.BlockSpec((AB, 1), lambda i: (i, 0)), blk_1n],
        out_specs=[pl.BlockSpec((AB, 1), lambda i: (i, 0)),
                   pl.BlockSpec((NBLKG, 1), lambda i: (0, 0))],
        out_shape=[jax.ShapeDtypeStruct((A, 1), jnp.int32),
                   jax.ShapeDtypeStruct((NBLKG, 1), jnp.int32)],
        scratch_shapes=[pltpu.VMEM((1, NE), f32)],
    )(eids, cnt)

    posf = pos.reshape(A)
    tok = jnp.arange(A, dtype=jnp.int32) // TOPK
    rows_sorted, gates_sorted = _sc_scatter(posf, tok, tv.reshape(A), A_PAD)
    xs = _sc_gather(xpk, rows_sorted)

    FFB = min(1024, FF)
    grid_spec = pltpu.PrefetchScalarGridSpec(
        num_scalar_prefetch=1,
        grid=(NBLKG, FF // FFB),
        in_specs=[pl.BlockSpec((BLKG, E // 2), lambda b, fb, m: (b, 0)),
                  pl.BlockSpec((BLKG, 1), lambda b, fb, m: (b, 0)),
                  pl.BlockSpec((1, E, FFB), lambda b, fb, m: (m[b], 0, fb)),
                  pl.BlockSpec((1, 1, FFB), lambda b, fb, m: (m[b], 0, fb)),
                  pl.BlockSpec((1, FFB, E), lambda b, fb, m: (m[b], fb, 0)),
                  pl.BlockSpec((1, 1, E), lambda b, fb, m: (m[b], 0, 0))],
        out_specs=pl.BlockSpec((BLKG, E // 2), lambda b, fb, m: (b, 0)),
        scratch_shapes=[pltpu.VMEM((BLKG, E), f32)],
    )
    ys = _pallas_call(
        functools.partial(_gffn_body, nfb=FF // FFB),
        grid_spec=grid_spec,
        out_shape=jax.ShapeDtypeStruct((A_PAD, E // 2), jnp.int32),
    )(b2e.reshape(NBLKG), xs, gates_sorted.reshape(A_PAD, 1),
      e_w1, e_b1.reshape(NE, 1, FF), e_w2, e_b2.reshape(NE, 1, E))

    perm = pos.reshape(T, TOPK).T.reshape(A)
    y3 = _sc_gather(ys, perm).reshape(TOPK, T, E // 2)

    out = _pallas_call(
        _fin_body,
        grid=(nt,),
        in_specs=[blk_te,
                  pl.BlockSpec((TOPK, TB, E // 2), lambda i: (0, i, 0)),
                  blk_1e, blk_1e],
        out_specs=blk_te,
        out_shape=te,
    )(x, y3, row(ln2_g), row(ln2_b))

    return out.reshape(B, S, E), aux.reshape(())
